# Initial kernel scaffold; baseline (speedup 1.0000x reference)
import jax
import jax.numpy as jnp
from jax.experimental import pallas as pl


def kernel(sparse_features, voxel_batch_idx, voxel_xy, dense_tensor, W_val, b_val, W_off, b_off, W_attn, b_attn, W_out, b_out):
    raise NotImplementedError("write your pallas kernel here")


# speedup vs baseline: 1.0558x; 1.0558x over previous
import functools

import jax
import jax.numpy as jnp
from jax import lax
from jax.experimental import pallas as pl
from jax.experimental.pallas import tpu as pltpu
from jax.experimental.pallas import tpu_sc as plsc

HEADS_ = 8
POINTS_ = 4


def _val_proj_body(d_ref, w_ref, b_ref, o_ref):
    o_ref[...] = lax.dot_general(
        d_ref[0], w_ref[...], (((0,), (0,)), ((), ())),
        preferred_element_type=jnp.float32) + b_ref[...][None, :]


def _val_proj(dense_flat, w_val, b_val):
    B, C, HW = dense_flat.shape
    MT = 1024
    gi = pl.cdiv(HW, MT)
    return pl.pallas_call(
        _val_proj_body,
        grid=(B, gi),
        in_specs=[
            pl.BlockSpec((1, C, MT), lambda b, i: (b, 0, i)),
            pl.BlockSpec((C, C), lambda b, i: (0, 0)),
            pl.BlockSpec((C,), lambda b, i: (0,)),
        ],
        out_specs=pl.BlockSpec((MT, C), lambda b, i: (b * gi + i, 0)),
        out_shape=jax.ShapeDtypeStruct((B * gi * MT, C), jnp.float32),
    )(dense_flat, w_val, b_val)


def _addr_body(HW, Hd, Wd, s_ref, x_ref, y_ref, b_ref, wo_ref, bo_ref,
               wa_ref, ba_ref, idx_ref, w_ref):
    s = s_ref[...]
    offm = lax.dot_general(s, wo_ref[...], (((1,), (0,)), ((), ())),
                           preferred_element_type=jnp.float32) + bo_ref[...][None, :]
    attn = lax.dot_general(s, wa_ref[...], (((1,), (0,)), ((), ())),
                           preferred_element_type=jnp.float32) + ba_ref[...][None, :]
    a = [attn[:, p * 8:(p + 1) * 8] for p in range(POINTS_)]
    m = jnp.maximum(jnp.maximum(a[0], a[1]), jnp.maximum(a[2], a[3]))
    e = [jnp.exp(v - m) for v in a]
    ssum = e[0] + e[1] + e[2] + e[3]
    aw = [v / ssum for v in e]

    xq = x_ref[...].astype(jnp.float32)
    yq = y_ref[...].astype(jnp.float32)
    bq = b_ref[...]
    TN = s.shape[0]
    h_arr = lax.broadcasted_iota(jnp.int32, (TN, 8), 1)
    ref_x = xq / Hd
    ref_y = yq / Wd
    idx_parts, w_parts = [], []
    for p in range(POINTS_):
        off_x = offm[:, p * 8:(p + 1) * 8]
        off_y = offm[:, 32 + p * 8:32 + (p + 1) * 8]
        ix = (ref_x + off_x / Wd) * Wd - 0.5
        iy = (ref_y + off_y / Hd) * Hd - 0.5
        x0 = jnp.floor(ix)
        y0 = jnp.floor(iy)
        wx1 = ix - x0
        wx0 = 1.0 - wx1
        wy1 = iy - y0
        wy0 = 1.0 - wy1
        for (yc, xc, wc) in ((y0, x0, wy0 * wx0), (y0, x0 + 1.0, wy0 * wx1),
                             (y0 + 1.0, x0, wy1 * wx0),
                             (y0 + 1.0, x0 + 1.0, wy1 * wx1)):
            inb = ((xc >= 0) & (xc <= Wd - 1) & (yc >= 0)
                   & (yc <= Hd - 1)).astype(jnp.float32)
            xi = jnp.clip(xc, 0, Wd - 1).astype(jnp.int32)
            yi = jnp.clip(yc, 0, Hd - 1).astype(jnp.int32)
            idx_parts.append(
                (bq * HW + yi * Wd + xi) * (HEADS_ // 2)
                + lax.shift_right_logical(h_arr, 1))
            w_parts.append(aw[p] * wc * inb)
    idx_ref[...] = jnp.concatenate(idx_parts, axis=1)
    w_ref[...] = jnp.concatenate(w_parts, axis=1)


def _addresses(sparse, xcol, ycol, bcol, w_off2, b_off2, w_attn2, b_attn2,
               HW, Hd, Wd):
    N, C = sparse.shape
    TN = 1000
    grid = N // TN
    return pl.pallas_call(
        functools.partial(_addr_body, HW, Hd, Wd),
        grid=(grid,),
        in_specs=[
            pl.BlockSpec((TN, C), lambda i: (i, 0)),
            pl.BlockSpec((TN, 1), lambda i: (i, 0)),
            pl.BlockSpec((TN, 1), lambda i: (i, 0)),
            pl.BlockSpec((TN, 1), lambda i: (i, 0)),
            pl.BlockSpec((C, 64), lambda i: (0, 0)),
            pl.BlockSpec((64,), lambda i: (0,)),
            pl.BlockSpec((C, 32), lambda i: (0, 0)),
            pl.BlockSpec((32,), lambda i: (0,)),
        ],
        out_specs=[
            pl.BlockSpec((TN, 128), lambda i: (i, 0)),
            pl.BlockSpec((TN, 128), lambda i: (i, 0)),
        ],
        out_shape=[
            jax.ShapeDtypeStruct((N, 128), jnp.int32),
            jax.ShapeDtypeStruct((N, 128), jnp.float32),
        ],
    )(sparse, xcol, ycol, bcol, w_off2, b_off2, w_attn2, b_attn2)


def _splat_lane(vec16, lane):
    idx = jnp.full((16, 1), lane, jnp.int32)
    return lax.gather(
        vec16, idx,
        dimension_numbers=lax.GatherDimensionNumbers(
            offset_dims=(), collapsed_slice_dims=(0,), start_index_map=(0,)),
        slice_sizes=(1,),
        mode=lax.GatherScatterMode.PROMISE_IN_BOUNDS)


def _gather_combine(table, cidx, cw, Np, dh):
    NW = 32
    NQW = Np // NW
    Q = 4
    NCH = NQW // Q
    R = Q * 128

    mesh = plsc.VectorSubcoreMesh(core_axis_name="c", subcore_axis_name="s")

    @functools.partial(
        pl.kernel, mesh=mesh,
        out_type=jax.ShapeDtypeStruct((Np * HEADS_, dh), jnp.float32),
        scratch_types=[
            pltpu.VMEM((Q, 128), jnp.int32),
            pltpu.VMEM((R,), jnp.float32),
            pltpu.VMEM((R, 2 * dh), jnp.float32),
            pltpu.VMEM((Q * HEADS_, dh), jnp.float32),
            pltpu.SemaphoreType.DMA,
        ],
    )
    def sc_kernel(table_hbm, idx_hbm, w_hbm, out_hbm, idx_v, w_v, rows_v,
                  out_v, sem):
        wid = lax.axis_index("s") * 2 + lax.axis_index("c")
        qw0 = wid * NQW

        def chunk_body(g, carry):
            q0 = qw0 + g * Q
            pltpu.sync_copy(idx_hbm.at[pl.ds(q0, Q)], idx_v)
            pltpu.sync_copy(w_hbm.at[pl.ds(q0 * 128, R)], w_v)
            handles = []
            for qi in range(Q):
                handles.append(pltpu.async_copy(
                    table_hbm.at[idx_v.at[qi]],
                    rows_v.at[pl.ds(qi * 128, 128)], sem))
            for hd in handles:
                hd.wait()
            for qi in range(Q):
                base = qi * 128

                def jj_body(jj, acc):
                    wb = base + jj * 16
                    w16 = w_v[pl.ds(wb, 16)]
                    acc = list(acc)
                    for k in range(2):
                        rb = wb + k * 8
                        for h in range(HEADS_):
                            wsp = _splat_lane(w16, k * 8 + h)
                            half = (h % 2) * dh
                            for gg in range(4):
                                acc[h * 4 + gg] = acc[h * 4 + gg] + wsp * \
                                    rows_v[rb + h, pl.ds(half + gg * 16, 16)]
                    return tuple(acc)

                acc0 = tuple(jnp.zeros((16,), jnp.float32)
                             for _ in range(HEADS_ * 4))
                acc = lax.fori_loop(0, POINTS_ * 2, jj_body, acc0)
                for h in range(HEADS_):
                    for gg in range(4):
                        out_v[qi * HEADS_ + h, pl.ds(gg * 16, 16)] = \
                            acc[h * 4 + gg]
            pltpu.sync_copy(out_v,
                            out_hbm.at[pl.ds(q0 * HEADS_, Q * HEADS_)])
            return carry

        lax.fori_loop(0, NCH, chunk_body, 0)

    return sc_kernel(table, cidx, cw)


def _out_proj_body(a_ref, w_ref, b_ref, s_ref, o_ref):
    o_ref[...] = s_ref[...] + lax.dot_general(
        a_ref[...], w_ref[...], (((1,), (0,)), ((), ())),
        preferred_element_type=jnp.float32) + b_ref[...][None, :]


def _out_proj(agg, w_out, b_out, sparse):
    N, C = sparse.shape
    TN = 1000
    return pl.pallas_call(
        _out_proj_body,
        grid=(N // TN,),
        in_specs=[
            pl.BlockSpec((TN, C), lambda i: (i, 0)),
            pl.BlockSpec((C, C), lambda i: (0, 0)),
            pl.BlockSpec((C,), lambda i: (0,)),
            pl.BlockSpec((TN, C), lambda i: (i, 0)),
        ],
        out_specs=pl.BlockSpec((TN, C), lambda i: (i, 0)),
        out_shape=jax.ShapeDtypeStruct((N, C), jnp.float32),
    )(agg, w_out, b_out, sparse)


def kernel(sparse_features, voxel_batch_idx, voxel_xy, dense_tensor,
           W_val, b_val, W_off, b_off, W_attn, b_attn, W_out, b_out):
    B, C, Hd, Wd = dense_tensor.shape
    N = sparse_features.shape[0]
    HW = Hd * Wd
    dh = C // HEADS_

    val = _val_proj(dense_tensor.reshape(B, C, HW), W_val, b_val)
    HWp = val.shape[0] // B
    table = val.reshape(B * HWp * (HEADS_ // 2), 2 * dh)

    W_off2 = W_off.reshape(C, HEADS_, POINTS_, 2).transpose(0, 3, 2, 1).reshape(C, 64)
    b_off2 = b_off.reshape(HEADS_, POINTS_, 2).transpose(2, 1, 0).reshape(64)
    W_attn2 = W_attn.reshape(C, HEADS_, POINTS_).transpose(0, 2, 1).reshape(C, 32)
    b_attn2 = b_attn.reshape(HEADS_, POINTS_).transpose(1, 0).reshape(32)
    xcol = voxel_xy[:, 0:1].astype(jnp.int32)
    ycol = voxel_xy[:, 1:2].astype(jnp.int32)
    bcol = voxel_batch_idx[:, None].astype(jnp.int32)
    cidx, cw = _addresses(sparse_features, xcol, ycol, bcol,
                          W_off2, b_off2, W_attn2, b_attn2, HWp, Hd, Wd)

    Np = ((N + 127) // 128) * 128
    cidx_p = jnp.pad(cidx, ((0, Np - N), (0, 0)))
    cw_p = jnp.pad(cw, ((0, Np - N), (0, 0))).reshape(Np * 128)
    agg = _gather_combine(table, cidx_p, cw_p, Np, dh)
    agg = agg[:N * HEADS_].reshape(N, C)

    return _out_proj(agg, W_out, b_out, sparse_features)


# speedup vs baseline: 1.1303x; 1.0706x over previous
import functools

import jax
import jax.numpy as jnp
from jax import lax
from jax.experimental import pallas as pl
from jax.experimental.pallas import tpu as pltpu
from jax.experimental.pallas import tpu_sc as plsc

HEADS_ = 8
POINTS_ = 4


def _val_proj_body(d_ref, w_ref, b_ref, o_ref):
    o_ref[...] = lax.dot_general(
        d_ref[0], w_ref[...], (((0,), (0,)), ((), ())),
        preferred_element_type=jnp.float32) + b_ref[...][None, :]


def _val_proj(dense_flat, w_val, b_val):
    B, C, HW = dense_flat.shape
    MT = 1024
    gi = pl.cdiv(HW, MT)
    return pl.pallas_call(
        _val_proj_body,
        grid=(B, gi),
        in_specs=[
            pl.BlockSpec((1, C, MT), lambda b, i: (b, 0, i)),
            pl.BlockSpec((C, C), lambda b, i: (0, 0)),
            pl.BlockSpec((C,), lambda b, i: (0,)),
        ],
        out_specs=pl.BlockSpec((MT, C), lambda b, i: (b * gi + i, 0)),
        out_shape=jax.ShapeDtypeStruct((B * gi * MT, C), jnp.float32),
    )(dense_flat, w_val, b_val)


def _addr_body(HW, Hd, Wd, s_ref, x_ref, y_ref, b_ref, wo_ref, bo_ref,
               wa_ref, ba_ref, idx_ref, w_ref):
    s = s_ref[...]
    offm = lax.dot_general(s, wo_ref[...], (((1,), (0,)), ((), ())),
                           preferred_element_type=jnp.float32) + bo_ref[...][None, :]
    attn = lax.dot_general(s, wa_ref[...], (((1,), (0,)), ((), ())),
                           preferred_element_type=jnp.float32) + ba_ref[...][None, :]
    a = [attn[:, p * 8:(p + 1) * 8] for p in range(POINTS_)]
    m = jnp.maximum(jnp.maximum(a[0], a[1]), jnp.maximum(a[2], a[3]))
    e = [jnp.exp(v - m) for v in a]
    ssum = e[0] + e[1] + e[2] + e[3]
    aw = [v / ssum for v in e]

    xq = x_ref[...].astype(jnp.float32)
    yq = y_ref[...].astype(jnp.float32)
    bq = b_ref[...]
    TN = s.shape[0]
    h_arr = lax.broadcasted_iota(jnp.int32, (TN, 8), 1)
    ref_x = xq / Hd
    ref_y = yq / Wd
    idx_parts, w_parts = [], []
    for p in range(POINTS_):
        off_x = offm[:, p * 8:(p + 1) * 8]
        off_y = offm[:, 32 + p * 8:32 + (p + 1) * 8]
        ix = (ref_x + off_x / Wd) * Wd - 0.5
        iy = (ref_y + off_y / Hd) * Hd - 0.5
        x0 = jnp.floor(ix)
        y0 = jnp.floor(iy)
        wx1 = ix - x0
        wx0 = 1.0 - wx1
        wy1 = iy - y0
        wy0 = 1.0 - wy1
        for (yc, xc, wc) in ((y0, x0, wy0 * wx0), (y0, x0 + 1.0, wy0 * wx1),
                             (y0 + 1.0, x0, wy1 * wx0),
                             (y0 + 1.0, x0 + 1.0, wy1 * wx1)):
            inb = ((xc >= 0) & (xc <= Wd - 1) & (yc >= 0)
                   & (yc <= Hd - 1)).astype(jnp.float32)
            xi = jnp.clip(xc, 0, Wd - 1).astype(jnp.int32)
            yi = jnp.clip(yc, 0, Hd - 1).astype(jnp.int32)
            idx_parts.append(
                (bq * HW + yi * Wd + xi) * (HEADS_ // 2)
                + lax.shift_right_logical(h_arr, 1))
            w_parts.append(aw[p] * wc * inb)
    idx_ref[...] = jnp.concatenate(idx_parts, axis=1)
    w_ref[...] = jnp.concatenate(w_parts, axis=1)


def _addresses(sparse, xcol, ycol, bcol, w_off2, b_off2, w_attn2, b_attn2,
               HW, Hd, Wd):
    N, C = sparse.shape
    TN = 1000
    grid = N // TN
    return pl.pallas_call(
        functools.partial(_addr_body, HW, Hd, Wd),
        grid=(grid,),
        in_specs=[
            pl.BlockSpec((TN, C), lambda i: (i, 0)),
            pl.BlockSpec((TN, 1), lambda i: (i, 0)),
            pl.BlockSpec((TN, 1), lambda i: (i, 0)),
            pl.BlockSpec((TN, 1), lambda i: (i, 0)),
            pl.BlockSpec((C, 64), lambda i: (0, 0)),
            pl.BlockSpec((64,), lambda i: (0,)),
            pl.BlockSpec((C, 32), lambda i: (0, 0)),
            pl.BlockSpec((32,), lambda i: (0,)),
        ],
        out_specs=[
            pl.BlockSpec((TN, 128), lambda i: (i, 0)),
            pl.BlockSpec((TN, 128), lambda i: (i, 0)),
        ],
        out_shape=[
            jax.ShapeDtypeStruct((N, 128), jnp.int32),
            jax.ShapeDtypeStruct((N, 128), jnp.float32),
        ],
    )(sparse, xcol, ycol, bcol, w_off2, b_off2, w_attn2, b_attn2)


def _splat_lane(vec16, lane):
    idx = jnp.full((16, 1), lane, jnp.int32)
    return lax.gather(
        vec16, idx,
        dimension_numbers=lax.GatherDimensionNumbers(
            offset_dims=(), collapsed_slice_dims=(0,), start_index_map=(0,)),
        slice_sizes=(1,),
        mode=lax.GatherScatterMode.PROMISE_IN_BOUNDS)


def _gather_combine(table, cidx, cw, Np, dh):
    NW = 32
    NQW = Np // NW
    Q = 2
    NCH = NQW // Q
    R = Q * 128

    mesh = plsc.VectorSubcoreMesh(core_axis_name="c", subcore_axis_name="s")

    @functools.partial(
        pl.kernel, mesh=mesh,
        out_type=jax.ShapeDtypeStruct((Np * HEADS_, dh), jnp.float32),
        scratch_types=[
            pltpu.VMEM((3, Q, 128), jnp.int32),
            pltpu.VMEM((3, Q, 128), jnp.float32),
            pltpu.VMEM((2, R, 2 * dh), jnp.float32),
            pltpu.VMEM((Q * HEADS_, dh), jnp.float32),
            pltpu.SemaphoreType.DMA,
            pltpu.SemaphoreType.DMA,
        ],
    )
    def sc_kernel(table_hbm, idx_hbm, w_hbm, out_hbm, idx_v, w_v, rows_v,
                  out_v, gsem, ssem):
        wid = lax.axis_index("s") * 2 + lax.axis_index("c")
        qw0 = wid * NQW

        pltpu.sync_copy(idx_hbm.at[pl.ds(qw0, Q)], idx_v.at[0])
        pltpu.sync_copy(w_hbm.at[pl.ds(qw0, Q)], w_v.at[0])

        def step(g, carry):
            gs = g % 2
            cs = 1 - gs
            s_in = g % 3
            s_nxt = (g + 1) % 3
            s_cmp = (g + 2) % 3
            gnxt = jnp.minimum(g + 1, NCH - 1)
            ccmp = jnp.maximum(g - 1, 0)
            handles = [
                pltpu.async_copy(table_hbm.at[idx_v.at[s_in, qi]],
                                 rows_v.at[gs, pl.ds(qi * 128, 128)], gsem)
                for qi in range(Q)]
            handles.append(pltpu.async_copy(
                idx_hbm.at[pl.ds(qw0 + gnxt * Q, Q)], idx_v.at[s_nxt], ssem))
            handles.append(pltpu.async_copy(
                w_hbm.at[pl.ds(qw0 + gnxt * Q, Q)], w_v.at[s_nxt], ssem))

            for qi in range(Q):
                base = qi * 128

                def jj_body(jj, acc):
                    wb = jj * 16
                    w16 = w_v[s_cmp, qi, pl.ds(wb, 16)]
                    acc = list(acc)
                    for k in range(2):
                        rb = base + wb + k * 8
                        for h in range(HEADS_):
                            wsp = _splat_lane(w16, k * 8 + h)
                            half = (h % 2) * dh
                            for gg in range(4):
                                acc[h * 4 + gg] = acc[h * 4 + gg] + wsp * \
                                    rows_v[cs, rb + h, pl.ds(half + gg * 16, 16)]
                    return tuple(acc)

                acc0 = tuple(jnp.zeros((16,), jnp.float32)
                             for _ in range(HEADS_ * 4))
                acc = lax.fori_loop(0, POINTS_ * 2, jj_body, acc0)
                for h in range(HEADS_):
                    for gg in range(4):
                        out_v[qi * HEADS_ + h, pl.ds(gg * 16, 16)] = \
                            acc[h * 4 + gg]
            pltpu.sync_copy(
                out_v, out_hbm.at[pl.ds((qw0 + ccmp * Q) * HEADS_, Q * HEADS_)])
            for hd in handles:
                hd.wait()
            return carry

        lax.fori_loop(0, NCH + 1, step, 0)

    return sc_kernel(table, cidx, cw)


def _out_proj_body(a_ref, w_ref, b_ref, s_ref, o_ref):
    o_ref[...] = s_ref[...] + lax.dot_general(
        a_ref[...], w_ref[...], (((1,), (0,)), ((), ())),
        preferred_element_type=jnp.float32) + b_ref[...][None, :]


def _out_proj(agg, w_out, b_out, sparse):
    N, C = sparse.shape
    TN = 1000
    return pl.pallas_call(
        _out_proj_body,
        grid=(N // TN,),
        in_specs=[
            pl.BlockSpec((TN, C), lambda i: (i, 0)),
            pl.BlockSpec((C, C), lambda i: (0, 0)),
            pl.BlockSpec((C,), lambda i: (0,)),
            pl.BlockSpec((TN, C), lambda i: (i, 0)),
        ],
        out_specs=pl.BlockSpec((TN, C), lambda i: (i, 0)),
        out_shape=jax.ShapeDtypeStruct((N, C), jnp.float32),
    )(agg, w_out, b_out, sparse)


def kernel(sparse_features, voxel_batch_idx, voxel_xy, dense_tensor,
           W_val, b_val, W_off, b_off, W_attn, b_attn, W_out, b_out):
    B, C, Hd, Wd = dense_tensor.shape
    N = sparse_features.shape[0]
    HW = Hd * Wd
    dh = C // HEADS_

    val = _val_proj(dense_tensor.reshape(B, C, HW), W_val, b_val)
    HWp = val.shape[0] // B
    table = val.reshape(B * HWp * (HEADS_ // 2), 2 * dh)

    W_off2 = W_off.reshape(C, HEADS_, POINTS_, 2).transpose(0, 3, 2, 1).reshape(C, 64)
    b_off2 = b_off.reshape(HEADS_, POINTS_, 2).transpose(2, 1, 0).reshape(64)
    W_attn2 = W_attn.reshape(C, HEADS_, POINTS_).transpose(0, 2, 1).reshape(C, 32)
    b_attn2 = b_attn.reshape(HEADS_, POINTS_).transpose(1, 0).reshape(32)
    xcol = voxel_xy[:, 0:1].astype(jnp.int32)
    ycol = voxel_xy[:, 1:2].astype(jnp.int32)
    bcol = voxel_batch_idx[:, None].astype(jnp.int32)
    cidx, cw = _addresses(sparse_features, xcol, ycol, bcol,
                          W_off2, b_off2, W_attn2, b_attn2, HWp, Hd, Wd)

    Np = ((N + 127) // 128) * 128
    cidx_p = jnp.pad(cidx, ((0, Np - N), (0, 0)))
    cw_p = jnp.pad(cw, ((0, Np - N), (0, 0)))
    agg = _gather_combine(table, cidx_p, cw_p, Np, dh)
    agg = agg[:N * HEADS_].reshape(N, C)

    return _out_proj(agg, W_out, b_out, sparse_features)
